# baseline (speedup 1.0000x reference)
import functools

import jax
import jax.numpy as jnp
from jax import lax
from jax.experimental import pallas as pl
from jax.experimental.pallas import tpu as pltpu
from jax.experimental.pallas import tpu_sc as plsc

N = 10000
E = 320000
NC = 2
NS = 16
NW = NC * NS
CHUNK = 128
NCHUNK = -(-E // (NW * CHUNK))
E_PAD = NW * NCHUNK * CHUNK
ACC_ROWS = 10240
ROWS_PER_SUB = ACC_ROWS // NS

_sc_mesh = plsc.VectorSubcoreMesh(core_axis_name="c", subcore_axis_name="s")


def _make_seg_sum(d):
    @functools.partial(
        pl.kernel,
        out_type=jax.ShapeDtypeStruct((NC, ACC_ROWS, d), jnp.float32),
        mesh=_sc_mesh,
        compiler_params=pltpu.CompilerParams(use_tc_tiling_on_sc=False),
        scratch_types=[
            pltpu.VMEM((NCHUNK, CHUNK), jnp.int32),
            pltpu.VMEM((NCHUNK, CHUNK), jnp.int32),
            pltpu.VMEM((CHUNK, d), jnp.float32),
            pltpu.VMEM_SHARED((ACC_ROWS, d), jnp.float32),
            pltpu.SemaphoreType.DMA,
        ],
    )
    def seg_sum(y_hbm, src_hbm, dst_hbm, out_hbm,
                src_v, dst_v, rows_v, acc_sh, sem):
        cid = lax.axis_index("c")
        sid = lax.axis_index("s")
        wid = sid * NC + cid
        row0 = sid * ROWS_PER_SUB
        @pl.loop(0, CHUNK)
        def _(r):
            for c in range(d // 16):
                rows_v.at[pl.ds(r, 1), pl.ds(16 * c, 16)][...] = (
                    jnp.zeros((1, 16), jnp.float32))

        for k in range(ROWS_PER_SUB // CHUNK):
            pltpu.sync_copy(rows_v, acc_sh.at[pl.ds(row0 + k * CHUNK, CHUNK)])
        pltpu.sync_copy(src_hbm.at[wid], src_v)
        pltpu.sync_copy(dst_hbm.at[wid], dst_v)
        plsc.subcore_barrier()

        @pl.loop(0, NCHUNK)
        def _(j):
            pltpu.async_copy(y_hbm.at[src_v.at[j]], rows_v, sem).wait()
            pltpu.sync_copy(rows_v, acc_sh.at[dst_v.at[j]], add=True)

        plsc.subcore_barrier()
        pltpu.sync_copy(acc_sh.at[pl.ds(row0, ROWS_PER_SUB)],
                        out_hbm.at[cid, pl.ds(row0, ROWS_PER_SUB)])

    return seg_sum


_seg_sum64 = _make_seg_sum(64)
_seg_sum32 = _make_seg_sum(32)

_BM = 1000


def _proj_body(x_ref, wn_ref, ws_ref, y_ref, xs_ref):
    xb = x_ref[...]
    y_ref[...] = jnp.dot(xb, wn_ref[...], preferred_element_type=jnp.float32)
    xs_ref[...] = jnp.dot(xb, ws_ref[...], preferred_element_type=jnp.float32)


def _tc_proj(x, w_nbr, w_self):
    din, dout = w_nbr.shape
    return pl.pallas_call(
        _proj_body,
        grid=(N // _BM,),
        in_specs=[
            pl.BlockSpec((_BM, din), lambda i: (i, 0)),
            pl.BlockSpec((din, dout), lambda i: (0, 0)),
            pl.BlockSpec((din, dout), lambda i: (0, 0)),
        ],
        out_specs=[
            pl.BlockSpec((_BM, dout), lambda i: (i, 0)),
            pl.BlockSpec((_BM, dout), lambda i: (i, 0)),
        ],
        out_shape=[
            jax.ShapeDtypeStruct((N, dout), jnp.float32),
            jax.ShapeDtypeStruct((N, dout), jnp.float32),
        ],
    )(x, w_nbr, w_self)


def _mid_body(xs_ref, p_ref, b_ref, wn_ref, ws_ref, y_ref, xs1_ref):
    h = jnp.maximum(xs_ref[...] + p_ref[0] + p_ref[1] + b_ref[...], 0.0)
    y_ref[...] = jnp.dot(h, wn_ref[...], preferred_element_type=jnp.float32)
    xs1_ref[...] = jnp.dot(h, ws_ref[...], preferred_element_type=jnp.float32)


def _tc_mid(xs, p, b, w_nbr, w_self):
    din, dout = w_nbr.shape
    return pl.pallas_call(
        _mid_body,
        grid=(N // _BM,),
        in_specs=[
            pl.BlockSpec((_BM, din), lambda i: (i, 0)),
            pl.BlockSpec((NC, _BM, din), lambda i: (0, i, 0)),
            pl.BlockSpec((1, din), lambda i: (0, 0)),
            pl.BlockSpec((din, dout), lambda i: (0, 0)),
            pl.BlockSpec((din, dout), lambda i: (0, 0)),
        ],
        out_specs=[
            pl.BlockSpec((_BM, dout), lambda i: (i, 0)),
            pl.BlockSpec((_BM, dout), lambda i: (i, 0)),
        ],
        out_shape=[
            jax.ShapeDtypeStruct((N, dout), jnp.float32),
            jax.ShapeDtypeStruct((N, dout), jnp.float32),
        ],
    )(xs, p, b.reshape(1, din), w_nbr, w_self)


def _out_body(xs_ref, q_ref, b_ref, o_ref):
    o_ref[...] = jnp.maximum(xs_ref[...] + q_ref[0] + q_ref[1] + b_ref[...], 0.0)


def _tc_out(xs, q, b):
    d = xs.shape[1]
    return pl.pallas_call(
        _out_body,
        grid=(N // _BM,),
        in_specs=[
            pl.BlockSpec((_BM, d), lambda i: (i, 0)),
            pl.BlockSpec((NC, _BM, d), lambda i: (0, i, 0)),
            pl.BlockSpec((1, d), lambda i: (0, 0)),
        ],
        out_specs=pl.BlockSpec((_BM, d), lambda i: (i, 0)),
        out_shape=jax.ShapeDtypeStruct((N, d), jnp.float32),
    )(xs, q, b.reshape(1, d))


def kernel(x, edge_index, W_self0, W_nbr0, b0, W_self1, W_nbr1, b1):
    pad = E_PAD - E
    src = jnp.concatenate(
        [edge_index[0].astype(jnp.int32), jnp.zeros((pad,), jnp.int32)]
    ).reshape(NW, NCHUNK, CHUNK)
    dst = jnp.concatenate(
        [edge_index[1].astype(jnp.int32),
         jnp.full((pad,), ACC_ROWS - 1, jnp.int32)]
    ).reshape(NW, NCHUNK, CHUNK)
    y0, xs0 = _tc_proj(x, W_nbr0, W_self0)
    p = _seg_sum64(y0, src, dst)
    y1, xs1 = _tc_mid(xs0, p, b0, W_nbr1, W_self1)
    q = _seg_sum32(y1, src, dst)
    return _tc_out(xs1, q, b1)


# speedup vs baseline: 1.0452x; 1.0452x over previous
import functools

import jax
import jax.numpy as jnp
from jax import lax
from jax.experimental import pallas as pl
from jax.experimental.pallas import tpu as pltpu
from jax.experimental.pallas import tpu_sc as plsc

N = 10000
E = 320000
NC = 2
NS = 16
NW = NC * NS
CHUNK = 128
NCHUNK = -(-E // (NW * CHUNK))
E_PAD = NW * NCHUNK * CHUNK
ACC_ROWS = 10240
ROWS_PER_SUB = ACC_ROWS // NS

_sc_mesh = plsc.VectorSubcoreMesh(core_axis_name="c", subcore_axis_name="s")


def _make_seg_sum(d):
    @functools.partial(
        pl.kernel,
        out_type=jax.ShapeDtypeStruct((NC, ACC_ROWS, d), jnp.float32),
        mesh=_sc_mesh,
        compiler_params=pltpu.CompilerParams(use_tc_tiling_on_sc=False),
        scratch_types=[
            pltpu.VMEM((NCHUNK, CHUNK), jnp.int32),
            pltpu.VMEM((NCHUNK, CHUNK), jnp.int32),
            pltpu.VMEM((CHUNK, d), jnp.float32),
            pltpu.VMEM_SHARED((ACC_ROWS, d), jnp.float32),
            pltpu.SemaphoreType.DMA,
        ],
    )
    def seg_sum(y_hbm, src_hbm, dst_hbm, zeros_hbm, out_hbm,
                src_v, dst_v, rows_v, acc_sh, sem):
        cid = lax.axis_index("c")
        sid = lax.axis_index("s")
        wid = sid * NC + cid
        row0 = sid * ROWS_PER_SUB
        pltpu.sync_copy(zeros_hbm.at[pl.ds(row0, ROWS_PER_SUB)],
                        acc_sh.at[pl.ds(row0, ROWS_PER_SUB)])
        pltpu.sync_copy(src_hbm.at[wid], src_v)
        pltpu.sync_copy(dst_hbm.at[wid], dst_v)
        plsc.subcore_barrier()

        @pl.loop(0, NCHUNK)
        def _(j):
            pltpu.async_copy(y_hbm.at[src_v.at[j]], rows_v, sem).wait()
            pltpu.sync_copy(rows_v, acc_sh.at[dst_v.at[j]], add=True)

        plsc.subcore_barrier()
        pltpu.sync_copy(acc_sh.at[pl.ds(row0, ROWS_PER_SUB)],
                        out_hbm.at[cid, pl.ds(row0, ROWS_PER_SUB)])

    return seg_sum


_seg_sum64 = _make_seg_sum(64)
_seg_sum32 = _make_seg_sum(32)

_BM = 1000


def _proj_body(x_ref, wn_ref, ws_ref, y_ref, xs_ref):
    xb = x_ref[...]
    y_ref[...] = jnp.dot(xb, wn_ref[...], preferred_element_type=jnp.float32)
    xs_ref[...] = jnp.dot(xb, ws_ref[...], preferred_element_type=jnp.float32)


def _tc_proj(x, w_nbr, w_self):
    din, dout = w_nbr.shape
    return pl.pallas_call(
        _proj_body,
        grid=(N // _BM,),
        in_specs=[
            pl.BlockSpec((_BM, din), lambda i: (i, 0)),
            pl.BlockSpec((din, dout), lambda i: (0, 0)),
            pl.BlockSpec((din, dout), lambda i: (0, 0)),
        ],
        out_specs=[
            pl.BlockSpec((_BM, dout), lambda i: (i, 0)),
            pl.BlockSpec((_BM, dout), lambda i: (i, 0)),
        ],
        out_shape=[
            jax.ShapeDtypeStruct((N, dout), jnp.float32),
            jax.ShapeDtypeStruct((N, dout), jnp.float32),
        ],
    )(x, w_nbr, w_self)


def _mid_body(xs_ref, p_ref, b_ref, wn_ref, ws_ref, y_ref, xs1_ref):
    h = jnp.maximum(xs_ref[...] + p_ref[0] + p_ref[1] + b_ref[...], 0.0)
    y_ref[...] = jnp.dot(h, wn_ref[...], preferred_element_type=jnp.float32)
    xs1_ref[...] = jnp.dot(h, ws_ref[...], preferred_element_type=jnp.float32)


def _tc_mid(xs, p, b, w_nbr, w_self):
    din, dout = w_nbr.shape
    return pl.pallas_call(
        _mid_body,
        grid=(N // _BM,),
        in_specs=[
            pl.BlockSpec((_BM, din), lambda i: (i, 0)),
            pl.BlockSpec((NC, _BM, din), lambda i: (0, i, 0)),
            pl.BlockSpec((1, din), lambda i: (0, 0)),
            pl.BlockSpec((din, dout), lambda i: (0, 0)),
            pl.BlockSpec((din, dout), lambda i: (0, 0)),
        ],
        out_specs=[
            pl.BlockSpec((_BM, dout), lambda i: (i, 0)),
            pl.BlockSpec((_BM, dout), lambda i: (i, 0)),
        ],
        out_shape=[
            jax.ShapeDtypeStruct((N, dout), jnp.float32),
            jax.ShapeDtypeStruct((N, dout), jnp.float32),
        ],
    )(xs, p, b.reshape(1, din), w_nbr, w_self)


def _out_body(xs_ref, q_ref, b_ref, o_ref):
    o_ref[...] = jnp.maximum(xs_ref[...] + q_ref[0] + q_ref[1] + b_ref[...], 0.0)


def _tc_out(xs, q, b):
    d = xs.shape[1]
    return pl.pallas_call(
        _out_body,
        grid=(N // _BM,),
        in_specs=[
            pl.BlockSpec((_BM, d), lambda i: (i, 0)),
            pl.BlockSpec((NC, _BM, d), lambda i: (0, i, 0)),
            pl.BlockSpec((1, d), lambda i: (0, 0)),
        ],
        out_specs=pl.BlockSpec((_BM, d), lambda i: (i, 0)),
        out_shape=jax.ShapeDtypeStruct((N, d), jnp.float32),
    )(xs, q, b.reshape(1, d))


def kernel(x, edge_index, W_self0, W_nbr0, b0, W_self1, W_nbr1, b1):
    pad = E_PAD - E
    src = jnp.concatenate(
        [edge_index[0].astype(jnp.int32), jnp.zeros((pad,), jnp.int32)]
    ).reshape(NW, NCHUNK, CHUNK)
    dst = jnp.concatenate(
        [edge_index[1].astype(jnp.int32),
         jnp.full((pad,), ACC_ROWS - 1, jnp.int32)]
    ).reshape(NW, NCHUNK, CHUNK)
    zeros64 = jnp.zeros((ACC_ROWS, 64), jnp.float32)
    zeros32 = jnp.zeros((ACC_ROWS, 32), jnp.float32)

    y0, xs0 = _tc_proj(x, W_nbr0, W_self0)
    p = _seg_sum64(y0, src, dst, zeros64)
    y1, xs1 = _tc_mid(xs0, p, b0, W_nbr1, W_self1)
    q = _seg_sum32(y1, src, dst, zeros32)
    return _tc_out(xs1, q, b1)
